# baseline (speedup 1.0000x reference)
import functools

import jax
import jax.numpy as jnp
from jax import lax
from jax.experimental import pallas as pl
from jax.experimental.pallas import tpu as pltpu
from jax.experimental.pallas import tpu_sc as plsc

D = 64
B = 16384
L = 16
NC = 2
NS = 16
NW = NC * NS
PER_W = B // NW
CHUNK = 64
NG = CHUNK // L
W = 8
NWAVE = CHUNK // W
N_CHUNKS = PER_W // CHUNK
NUM_ENT = 1000000
NUM_REL = 1000


def _make_sc_kernel():
    mesh = plsc.VectorSubcoreMesh(core_axis_name="c", subcore_axis_name="s")

    blk = lambda: pltpu.VMEM((W, 8, D), jnp.float32)
    @functools.partial(
        pl.kernel,
        out_type=jax.ShapeDtypeStruct((NW, L), jnp.float32),
        mesh=mesh,
        compiler_params=pltpu.CompilerParams(needs_layout_passes=False),
        scratch_types=[
            pltpu.VMEM((CHUNK,), jnp.int32),
            pltpu.VMEM((CHUNK,), jnp.int32),
            pltpu.VMEM((CHUNK,), jnp.int32),
            pltpu.VMEM((CHUNK,), jnp.int32),
            pltpu.VMEM((CHUNK, 2 * D), jnp.float32),
            pltpu.VMEM((CHUNK, 2 * D), jnp.float32),
            blk(), blk(), blk(), blk(),
            blk(), blk(), blk(), blk(),
            pltpu.VMEM((L,), jnp.float32),
            pltpu.SemaphoreType.DMA,
            pltpu.SemaphoreType.DMA,
            pltpu.SemaphoreType.DMA,
        ],
    )
    def sc_kernel(heads, rels, tails, ere3, eim3, rre, rim, out,
                  hidx, ridx, tidx, rdr, rbre, rbim,
                  a0, a1, a2, a3, b0, b1, b2, b3,
                  accv, semr, sema, semb):
        wid = lax.axis_index("s") * NC + lax.axis_index("c")
        base = wid * PER_W
        sets = [(a0, a1, a2, a3), (b0, b1, b2, b3)]
        sems = [sema, semb]

        def chunk_body(ck, accs):
            off = base + ck * CHUNK
            pltpu.sync_copy(heads.at[pl.ds(off, CHUNK)], hidx)
            pltpu.sync_copy(rels.at[pl.ds(off, CHUNK)], ridx)
            pltpu.sync_copy(tails.at[pl.ds(off, CHUNK)], tidx)
            for g in range(NG):
                sl = pl.ds(g * L, L)
                rdr[sl] = lax.shift_right_logical(ridx[sl], 1)
            cr1 = pltpu.async_copy(rre.at[rdr], rbre, semr)
            cr2 = pltpu.async_copy(rim.at[rdr], rbim, semr)
            cr1.wait()
            cr2.wait()

            hblk, hsub, tblk, tsub, rhalf = [], [], [], [], []
            for g in range(NG):
                sl = pl.ds(g * L, L)
                hv = hidx[sl]
                tv = tidx[sl]
                rv = ridx[sl]
                hblk.append(lax.shift_right_logical(hv, 3))
                tblk.append(lax.shift_right_logical(tv, 3))
                hsub.append(lax.bitwise_and(hv, 7))
                tsub.append(lax.bitwise_and(tv, 7))
                rhalf.append(lax.bitwise_and(rv, 1) * D)

            def fire(w):
                g, h = w // 2, w % 2
                s = sets[w % 2]
                sem = sems[w % 2]
                for l in range(W):
                    li = h * W + l
                    pltpu.async_copy(ere3.at[hblk[g][li]], s[0].at[l], sem)
                    pltpu.async_copy(eim3.at[hblk[g][li]], s[1].at[l], sem)
                    pltpu.async_copy(ere3.at[tblk[g][li]], s[2].at[l], sem)
                    pltpu.async_copy(eim3.at[tblk[g][li]], s[3].at[l], sem)

            def drain(w):
                s = sets[w % 2]
                sem = sems[w % 2]
                for l in range(W):
                    for r in s:
                        pltpu.make_async_copy(
                            ere3.at[0], r.at[l], sem).wait()

            fire(0)
            new = list(accs)
            for w in range(NWAVE):
                if w + 1 < NWAVE:
                    fire(w + 1)
                drain(w)
                g, h = w // 2, w % 2
                s = sets[w % 2]
                for l in range(W):
                    li = h * W + l
                    t = w * W + l
                    hs = hsub[g][li]
                    ts = tsub[g][li]
                    rh = rhalf[g][li]
                    for j in range(D // L):
                        dsl = pl.ds(j * L, L)
                        vhr = s[0][l, hs, dsl]
                        vhi = s[1][l, hs, dsl]
                        vtr = s[2][l, ts, dsl]
                        vti = s[3][l, ts, dsl]
                        rsl = pl.ds(rh + j * L, L)
                        vrr = rbre[t, rsl]
                        vri = rbim[t, rsl]
                        new[j] = (new[j] + vrr * (vhr * vtr + vhi * vti)
                                  + vri * (vhr * vti - vhi * vtr))
            return tuple(new)

        accs = lax.fori_loop(
            0, N_CHUNKS, chunk_body,
            tuple(jnp.zeros((L,), jnp.float32) for _ in range(D // L)))
        total = accs[0]
        for j in range(1, D // L):
            total = total + accs[j]
        accv[...] = total
        pltpu.sync_copy(accv, out.at[wid])

    return sc_kernel


_sc_score = _make_sc_kernel()


def kernel(heads, rels, tails, entity_re, entity_im, r_re, r_im):
    parts = _sc_score(
        heads.astype(jnp.int32),
        rels.astype(jnp.int32),
        tails.astype(jnp.int32),
        entity_re.reshape(NUM_ENT // 8, 8, D),
        entity_im.reshape(NUM_ENT // 8, 8, D),
        r_re.reshape(NUM_REL // 2, 2 * D),
        r_im.reshape(NUM_REL // 2, 2 * D),
    )
    return jnp.sum(parts)


# speedup vs baseline: 1.0083x; 1.0083x over previous
import functools

import jax
import jax.numpy as jnp
from jax import lax
from jax.experimental import pallas as pl
from jax.experimental.pallas import tpu as pltpu
from jax.experimental.pallas import tpu_sc as plsc

D = 64
B = 16384
L = 16
NC = 2
NS = 16
NW = NC * NS
PER_W = B // NW
CHUNK = 64
NG = CHUNK // L
W = 8
NWAVE = CHUNK // W
N_CHUNKS = PER_W // CHUNK
NUM_ENT = 1000000
NUM_REL = 1000


def _make_sc_kernel():
    mesh = plsc.VectorSubcoreMesh(core_axis_name="c", subcore_axis_name="s")

    blk = lambda: pltpu.VMEM((W, 8, D), jnp.float32)
    @functools.partial(
        pl.kernel,
        out_type=jax.ShapeDtypeStruct((NW, L), jnp.float32),
        mesh=mesh,
        compiler_params=pltpu.CompilerParams(needs_layout_passes=False),
        scratch_types=[
            pltpu.VMEM((CHUNK,), jnp.int32),
            pltpu.VMEM((CHUNK,), jnp.int32),
            pltpu.VMEM((CHUNK,), jnp.int32),
            pltpu.VMEM((CHUNK,), jnp.int32),
            pltpu.VMEM((CHUNK, 2 * D), jnp.float32),
            pltpu.VMEM((CHUNK, 2 * D), jnp.float32),
            blk(), blk(), blk(), blk(),
            blk(), blk(), blk(), blk(),
            pltpu.VMEM((L,), jnp.float32),
            pltpu.SemaphoreType.DMA,
            pltpu.SemaphoreType.DMA,
            pltpu.SemaphoreType.DMA,
        ],
    )
    def sc_kernel(heads, rels, tails, ere3, eim3, rre, rim, out,
                  hidx, ridx, tidx, rdr, rbre, rbim,
                  a0, a1, a2, a3, b0, b1, b2, b3,
                  accv, semr, sema, semb):
        wid = lax.axis_index("s") * NC + lax.axis_index("c")
        base = wid * PER_W
        sets = [(a0, a1, a2, a3), (b0, b1, b2, b3)]
        sems = [sema, semb]

        def chunk_body(ck, accs):
            off = base + ck * CHUNK
            pltpu.sync_copy(heads.at[pl.ds(off, CHUNK)], hidx)
            pltpu.sync_copy(rels.at[pl.ds(off, CHUNK)], ridx)
            pltpu.sync_copy(tails.at[pl.ds(off, CHUNK)], tidx)
            for g in range(NG):
                sl = pl.ds(g * L, L)
                rdr[sl] = lax.shift_right_logical(ridx[sl], 1)
            cr1 = pltpu.async_copy(rre.at[rdr], rbre, semr)
            cr2 = pltpu.async_copy(rim.at[rdr], rbim, semr)
            cr1.wait()
            cr2.wait()

            hblk, hsub, tblk, tsub, rhalf = [], [], [], [], []
            for g in range(NG):
                sl = pl.ds(g * L, L)
                hv = hidx[sl]
                tv = tidx[sl]
                rv = ridx[sl]
                hblk.append(lax.shift_right_logical(hv, 3))
                tblk.append(lax.shift_right_logical(tv, 3))
                hsub.append(lax.bitwise_and(hv, 7))
                tsub.append(lax.bitwise_and(tv, 7))
                rhalf.append(lax.bitwise_and(rv, 1) * D)

            def fire(w):
                g, h = w // 2, w % 2
                s = sets[w % 2]
                sem = sems[w % 2]
                for l in range(W):
                    li = h * W + l
                    pltpu.async_copy(ere3.at[hblk[g][li]], s[0].at[l], sem)
                    pltpu.async_copy(eim3.at[hblk[g][li]], s[1].at[l], sem)
                    pltpu.async_copy(ere3.at[tblk[g][li]], s[2].at[l], sem)
                    pltpu.async_copy(eim3.at[tblk[g][li]], s[3].at[l], sem)

            def drain(w):
                s = sets[w % 2]
                sem = sems[w % 2]
                for r in s:
                    pltpu.make_async_copy(
                        ere3.at[pl.ds(0, W)], r, sem).wait()

            fire(0)
            new = list(accs)
            for w in range(NWAVE):
                if w + 1 < NWAVE:
                    fire(w + 1)
                drain(w)
                g, h = w // 2, w % 2
                s = sets[w % 2]
                for l in range(W):
                    li = h * W + l
                    t = w * W + l
                    hs = hsub[g][li]
                    ts = tsub[g][li]
                    rh = rhalf[g][li]
                    for j in range(D // L):
                        dsl = pl.ds(j * L, L)
                        vhr = s[0][l, hs, dsl]
                        vhi = s[1][l, hs, dsl]
                        vtr = s[2][l, ts, dsl]
                        vti = s[3][l, ts, dsl]
                        rsl = pl.ds(rh + j * L, L)
                        vrr = rbre[t, rsl]
                        vri = rbim[t, rsl]
                        new[j] = (new[j] + vrr * (vhr * vtr + vhi * vti)
                                  + vri * (vhr * vti - vhi * vtr))
            return tuple(new)

        accs = lax.fori_loop(
            0, N_CHUNKS, chunk_body,
            tuple(jnp.zeros((L,), jnp.float32) for _ in range(D // L)))
        total = accs[0]
        for j in range(1, D // L):
            total = total + accs[j]
        accv[...] = total
        pltpu.sync_copy(accv, out.at[wid])

    return sc_kernel


_sc_score = _make_sc_kernel()


def kernel(heads, rels, tails, entity_re, entity_im, r_re, r_im):
    parts = _sc_score(
        heads.astype(jnp.int32),
        rels.astype(jnp.int32),
        tails.astype(jnp.int32),
        entity_re.reshape(NUM_ENT // 8, 8, D),
        entity_im.reshape(NUM_ENT // 8, 8, D),
        r_re.reshape(NUM_REL // 2, 2 * D),
        r_im.reshape(NUM_REL // 2, 2 * D),
    )
    return jnp.sum(parts)
